# Initial kernel scaffold; baseline (speedup 1.0000x reference)
import jax
import jax.numpy as jnp
from jax.experimental import pallas as pl


def kernel(e_emb, b_emb, s_emb, edge_index, We, be, Wb, bb, Ws, bs, b_curvature, s_curvature):
    raise NotImplementedError("write your pallas kernel here")


# speedup vs baseline: 4.7143x; 4.7143x over previous
import jax
import jax.numpy as jnp
from jax import lax
from jax.experimental import pallas as pl
from jax.experimental.pallas import tpu as pltpu
from jax.experimental.pallas import tpu_sc as plsc

_N = 10000
_D = 128
_NC = 2
_NS = 16
_E = 320000
_PT = _E // (_NC * _NS)
_CH = 80
_NCH = _PT // _CH
_SLAB = 624
_SLAB_LAST = _N - (_NS - 1) * _SLAB
_LAST0 = (_NS - 1) * _SLAB
_BN = 1000
_EPS = 1e-12


def _rownorm(x):
    return jnp.sqrt(jnp.sum(x * x, axis=1, keepdims=True))


def _matmul_t(x, w):
    return lax.dot_general(x, w, (((1,), (1,)), ((), ())),
                           preferred_element_type=jnp.float32)


def _log_map_table(b, sc):
    bn = _rownorm(b)
    at = 0.5 * jnp.log((1.0 + sc * bn) / (1.0 - sc * bn))
    return (2.0 / sc) * at * b / bn


def _sphere_table(s, ws, bs):
    ns = s / jnp.maximum(_rownorm(s), _EPS)
    ts0 = _matmul_t(ns, ws) + bs
    return ts0 / jnp.maximum(_rownorm(ts0), _EPS)


def _tc_prologue_body(cb_ref, b_ref, s_ref, ws_ref, bs_ref, tb_ref, ts_ref):
    sc = jnp.sqrt(cb_ref[0, 0])
    tb_ref[...] = _log_map_table(b_ref[...], sc)
    ts_ref[...] = _sphere_table(s_ref[...], ws_ref[...], bs_ref[...])


def _update_from_sums(se, sb, ss, degc, cb_ref, we_ref, be_ref, wb_ref, bb_ref):
    sc = jnp.sqrt(cb_ref[0, 0])
    invd = 1.0 / jnp.maximum(degc, 1.0)
    mask = jnp.minimum(degc, 1.0)
    ue = _matmul_t(se * invd, we_ref[...]) + be_ref[...] * mask
    e_new = jnp.where(ue >= 0, ue, 0.2 * ue)
    nb = _matmul_t(sb * invd, wb_ref[...]) + bb_ref[...] * mask
    nbn = _rownorm(nb)
    b_new = jnp.tanh(sc * nbn * 0.5) * nb / (sc * nbn)
    mts = ss * invd
    s_new = mts / jnp.maximum(_rownorm(mts), _EPS)
    return e_new, b_new, s_new


def _sum2(ref):
    return ref[0] + ref[1]


def _tc_mid_body(cb_ref, se_ref, sb_ref, ss_ref, deg_ref, we_ref, be_ref,
                 wb_ref, bb_ref, wsn_ref, bsn_ref, e_ref, tb_ref, ts_ref):
    degc = deg_ref[0, :, 0:1] + deg_ref[1, :, 0:1]
    e_new, b_new, s_new = _update_from_sums(
        _sum2(se_ref), _sum2(sb_ref), _sum2(ss_ref), degc,
        cb_ref, we_ref, be_ref, wb_ref, bb_ref)
    sc = jnp.sqrt(cb_ref[0, 0])
    e_ref[...] = e_new
    tb_ref[...] = _log_map_table(b_new, sc)
    ts_ref[...] = _sphere_table(s_new, wsn_ref[...], bsn_ref[...])


def _tc_epilogue_body(cb_ref, se_ref, sb_ref, ss_ref, deg_ref, we_ref, be_ref,
                      wb_ref, bb_ref, e_ref, b_ref, s_ref):
    degc = deg_ref[0, :, 0:1] + deg_ref[1, :, 0:1]
    e_new, b_new, s_new = _update_from_sums(
        _sum2(se_ref), _sum2(sb_ref), _sum2(ss_ref), degc,
        cb_ref, we_ref, be_ref, wb_ref, bb_ref)
    e_ref[...] = e_new
    b_ref[...] = b_new
    s_ref[...] = s_new


_GRID = _N // _BN

_SCALAR_SPEC = pl.BlockSpec((1, 1), lambda i: (0, 0), memory_space=pltpu.SMEM)
_ROW_SPEC = pl.BlockSpec((_BN, _D), lambda i: (i, 0))
_S_SPEC = pl.BlockSpec((_NC, _BN, _D), lambda i: (0, i, 0))
_DEG_SPEC = pl.BlockSpec((_NC, _BN, 16), lambda i: (0, i, 0))
_W_SPEC = pl.BlockSpec((_D, _D), lambda i: (0, 0))
_B_SPEC = pl.BlockSpec((1, _D), lambda i: (0, 0))

_ROW_SHAPE = jax.ShapeDtypeStruct((_N, _D), jnp.float32)


def _tc_prologue(cb, b, s, ws0, bs0):
    return pl.pallas_call(
        _tc_prologue_body,
        grid=(_GRID,),
        in_specs=[_SCALAR_SPEC, _ROW_SPEC, _ROW_SPEC, _W_SPEC, _B_SPEC],
        out_specs=[_ROW_SPEC, _ROW_SPEC],
        out_shape=[_ROW_SHAPE, _ROW_SHAPE],
    )(cb, b, s, ws0, bs0)


def _tc_mid(cb, se, sb, ss, deg2, we, be, wb, bb, wsn, bsn):
    return pl.pallas_call(
        _tc_mid_body,
        grid=(_GRID,),
        in_specs=[_SCALAR_SPEC, _S_SPEC, _S_SPEC, _S_SPEC, _DEG_SPEC,
                  _W_SPEC, _B_SPEC, _W_SPEC, _B_SPEC, _W_SPEC, _B_SPEC],
        out_specs=[_ROW_SPEC, _ROW_SPEC, _ROW_SPEC],
        out_shape=[_ROW_SHAPE, _ROW_SHAPE, _ROW_SHAPE],
    )(cb, se, sb, ss, deg2, we, be, wb, bb, wsn, bsn)


def _tc_epilogue(cb, se, sb, ss, deg2, we, be, wb, bb):
    return pl.pallas_call(
        _tc_epilogue_body,
        grid=(_GRID,),
        in_specs=[_SCALAR_SPEC, _S_SPEC, _S_SPEC, _S_SPEC, _DEG_SPEC,
                  _W_SPEC, _B_SPEC, _W_SPEC, _B_SPEC],
        out_specs=[_ROW_SPEC, _ROW_SPEC, _ROW_SPEC],
        out_shape=[_ROW_SHAPE, _ROW_SHAPE, _ROW_SHAPE],
    )(cb, se, sb, ss, deg2, we, be, wb, bb)


def _zero_slab(s, zsrc_hbm, acc_sh, r0):
    @pl.when(s < _NS - 1)
    def _():
        pltpu.sync_copy(zsrc_hbm.at[pl.ds(0, _SLAB)],
                        acc_sh.at[pl.ds(r0, _SLAB)])

    @pl.when(s == _NS - 1)
    def _():
        pltpu.sync_copy(zsrc_hbm, acc_sh.at[pl.ds(_LAST0, _SLAB_LAST)])


def _write_slab(s, acc_sh, out_hbm, c, r0):
    @pl.when(s < _NS - 1)
    def _():
        pltpu.sync_copy(acc_sh.at[pl.ds(r0, _SLAB)],
                        out_hbm.at[c, pl.ds(r0, _SLAB)])

    @pl.when(s == _NS - 1)
    def _():
        pltpu.sync_copy(acc_sh.at[pl.ds(_LAST0, _SLAB_LAST)],
                        out_hbm.at[c, pl.ds(_LAST0, _SLAB_LAST)])


def _make_sc_segsum(compute_deg):
    mesh = plsc.VectorSubcoreMesh(core_axis_name="c", subcore_axis_name="s",
                                  num_cores=_NC, num_subcores=_NS)

    out_type = [jax.ShapeDtypeStruct((_NC, _N, _D), jnp.float32)]
    scratch = [
        pltpu.VMEM_SHARED((_N, _D), jnp.float32),
        pltpu.VMEM((_CH,), jnp.int32),
        pltpu.VMEM((_CH,), jnp.int32),
        pltpu.VMEM((_CH,), jnp.int32),
        pltpu.VMEM((_CH,), jnp.int32),
        pltpu.VMEM((_CH, _D), jnp.float32),
        pltpu.VMEM((_CH, _D), jnp.float32),
        pltpu.SemaphoreType.DMA,
        pltpu.SemaphoreType.DMA,
    ]
    if compute_deg:
        out_type.append(jax.ShapeDtypeStruct((_NC, _N, 16), jnp.float32))
        scratch += [
            pltpu.VMEM_SHARED((_N, 16), jnp.float32),
            pltpu.VMEM((_CH, 16), jnp.float32),
        ]

    def common(t_hbm, src_hbm, dst_hbm, zrows_hbm, zdeg_hbm, ones_hbm,
               out_hbm, deg_hbm, acc_sh, srcb0, dstb0, srcb1, dstb1,
               rows0, rows1, sem0, sem1, deg_sh, onesb):
        c = lax.axis_index("c")
        s = lax.axis_index("s")
        r0 = pl.multiple_of(s * _SLAB, 8)

        _zero_slab(s, zrows_hbm, acc_sh, r0)
        if compute_deg:
            _zero_slab(s, zdeg_hbm, deg_sh, r0)
            pltpu.sync_copy(ones_hbm, onesb)
        plsc.subcore_barrier()

        base0 = (c * _NS + s) * _PT

        def start(j, srcb, dstb, rows, sem):
            base = base0 + j * _CH
            pltpu.sync_copy(src_hbm.at[pl.ds(base, _CH)], srcb)
            pltpu.sync_copy(dst_hbm.at[pl.ds(base, _CH)], dstb)
            pltpu.async_copy(t_hbm.at[srcb], rows, sem)

        def finish(srcb, dstb, rows, sem):
            pltpu.make_async_copy(t_hbm.at[srcb], rows, sem).wait()
            pltpu.sync_copy(rows, acc_sh.at[dstb], add=True)
            if compute_deg:
                pltpu.sync_copy(onesb, deg_sh.at[dstb], add=True)

        start(0, srcb0, dstb0, rows0, sem0)

        def pair(i, carry):
            j = 2 * i
            start(j + 1, srcb1, dstb1, rows1, sem1)
            finish(srcb0, dstb0, rows0, sem0)
            start(j + 2, srcb0, dstb0, rows0, sem0)
            finish(srcb1, dstb1, rows1, sem1)
            return carry

        lax.fori_loop(0, (_NCH - 1) // 2, pair, 0)
        finish(srcb0, dstb0, rows0, sem0)

        plsc.subcore_barrier()
        _write_slab(s, acc_sh, out_hbm, c, r0)
        if compute_deg:
            _write_slab(s, deg_sh, deg_hbm, c, r0)

    if compute_deg:
        def body(t_hbm, src_hbm, dst_hbm, zrows_hbm, zdeg_hbm, ones_hbm,
                 out_hbm, deg_hbm, acc_sh, srcb0, dstb0, srcb1, dstb1,
                 rows0, rows1, sem0, sem1, deg_sh, onesb):
            common(t_hbm, src_hbm, dst_hbm, zrows_hbm, zdeg_hbm, ones_hbm,
                   out_hbm, deg_hbm, acc_sh, srcb0, dstb0, srcb1, dstb1,
                   rows0, rows1, sem0, sem1, deg_sh, onesb)
    else:
        def body(t_hbm, src_hbm, dst_hbm, zrows_hbm, zdeg_hbm, ones_hbm,
                 out_hbm, acc_sh, srcb0, dstb0, srcb1, dstb1,
                 rows0, rows1, sem0, sem1):
            common(t_hbm, src_hbm, dst_hbm, zrows_hbm, zdeg_hbm, ones_hbm,
                   out_hbm, None, acc_sh, srcb0, dstb0, srcb1, dstb1,
                   rows0, rows1, sem0, sem1, None, None)

    return pl.kernel(body, out_type=out_type, mesh=mesh,
                     scratch_types=scratch,
                     compiler_params=pltpu.CompilerParams(
                         use_tc_tiling_on_sc=False))


def _sc_factory():
    return _make_sc_segsum(True), _make_sc_segsum(False)


def kernel(e_emb, b_emb, s_emb, edge_index, We, be, Wb, bb, Ws, bs,
           b_curvature, s_curvature):
    sc_deg, sc_plain = _sc_factory()
    src = edge_index[0].astype(jnp.int32)
    dst = edge_index[1].astype(jnp.int32)
    cb = b_curvature.reshape(1, 1)
    be2 = be.reshape(2, 1, _D)
    bb2 = bb.reshape(2, 1, _D)
    bs2 = bs.reshape(2, 1, _D)
    zrows = jnp.zeros((_SLAB_LAST, _D), jnp.float32)
    zdeg = jnp.zeros((_SLAB_LAST, 16), jnp.float32)
    ones = jnp.ones((_CH, 16), jnp.float32)

    tb0, ts0 = _tc_prologue(cb, b_emb, s_emb, Ws[0], bs2[0])
    se0, deg2 = sc_deg(e_emb, src, dst, zrows, zdeg, ones)
    sb0 = sc_plain(tb0, src, dst, zrows, zdeg, ones)
    ss0 = sc_plain(ts0, src, dst, zrows, zdeg, ones)
    if isinstance(sb0, (list, tuple)):
        sb0, ss0 = sb0[0], ss0[0]
    e1, tb1, ts1 = _tc_mid(cb, se0, sb0, ss0, deg2,
                           We[0], be2[0], Wb[0], bb2[0], Ws[1], bs2[1])
    se1 = sc_plain(e1, src, dst, zrows, zdeg, ones)
    sb1 = sc_plain(tb1, src, dst, zrows, zdeg, ones)
    ss1 = sc_plain(ts1, src, dst, zrows, zdeg, ones)
    if isinstance(se1, (list, tuple)):
        se1, sb1, ss1 = se1[0], sb1[0], ss1[0]
    e2, b2, s2 = _tc_epilogue(cb, se1, sb1, ss1, deg2,
                              We[1], be2[1], Wb[1], bb2[1])
    return (e2, b2, s2)


# speedup vs baseline: 6.9139x; 1.4666x over previous
import jax
import jax.numpy as jnp
from jax import lax
from jax.experimental import pallas as pl
from jax.experimental.pallas import tpu as pltpu
from jax.experimental.pallas import tpu_sc as plsc

_N = 10000
_D = 128
_NC = 2
_NS = 16
_E = 320000
_PT = _E // (_NC * _NS)
_CH = 80
_NCH = _PT // _CH
_SLAB = 624
_SLAB_LAST = _N - (_NS - 1) * _SLAB
_LAST0 = (_NS - 1) * _SLAB
_BN = 1000
_EPS = 1e-12


def _rownorm(x):
    return jnp.sqrt(jnp.sum(x * x, axis=1, keepdims=True))


def _matmul_t(x, w):
    return lax.dot_general(x, w, (((1,), (1,)), ((), ())),
                           preferred_element_type=jnp.float32)


def _log_map_table(b, sc):
    bn = _rownorm(b)
    at = 0.5 * jnp.log((1.0 + sc * bn) / (1.0 - sc * bn))
    return (2.0 / sc) * at * b / bn


def _sphere_table(s, ws, bs):
    ns = s / jnp.maximum(_rownorm(s), _EPS)
    ts0 = _matmul_t(ns, ws) + bs
    return ts0 / jnp.maximum(_rownorm(ts0), _EPS)


def _tc_prologue_body(cb_ref, b_ref, s_ref, ws_ref, bs_ref, tb_ref, ts_ref):
    sc = jnp.sqrt(cb_ref[0, 0])
    tb_ref[...] = _log_map_table(b_ref[...], sc)
    ts_ref[...] = _sphere_table(s_ref[...], ws_ref[...], bs_ref[...])


def _update_from_sums(se, sb, ss, degc, cb_ref, we_ref, be_ref, wb_ref, bb_ref):
    sc = jnp.sqrt(cb_ref[0, 0])
    invd = 1.0 / jnp.maximum(degc, 1.0)
    mask = jnp.minimum(degc, 1.0)
    ue = _matmul_t(se * invd, we_ref[...]) + be_ref[...] * mask
    e_new = jnp.where(ue >= 0, ue, 0.2 * ue)
    nb = _matmul_t(sb * invd, wb_ref[...]) + bb_ref[...] * mask
    nbn = _rownorm(nb)
    b_new = jnp.tanh(sc * nbn * 0.5) * nb / (sc * nbn)
    mts = ss * invd
    s_new = mts / jnp.maximum(_rownorm(mts), _EPS)
    return e_new, b_new, s_new


def _sum2(ref):
    return ref[0] + ref[1]


def _tc_mid_body(cb_ref, se_ref, sb_ref, ss_ref, deg_ref, we_ref, be_ref,
                 wb_ref, bb_ref, wsn_ref, bsn_ref, e_ref, tb_ref, ts_ref):
    degc = deg_ref[0, :, 0:1] + deg_ref[1, :, 0:1]
    e_new, b_new, s_new = _update_from_sums(
        _sum2(se_ref), _sum2(sb_ref), _sum2(ss_ref), degc,
        cb_ref, we_ref, be_ref, wb_ref, bb_ref)
    sc = jnp.sqrt(cb_ref[0, 0])
    e_ref[...] = e_new
    tb_ref[...] = _log_map_table(b_new, sc)
    ts_ref[...] = _sphere_table(s_new, wsn_ref[...], bsn_ref[...])


def _tc_epilogue_body(cb_ref, se_ref, sb_ref, ss_ref, deg_ref, we_ref, be_ref,
                      wb_ref, bb_ref, e_ref, b_ref, s_ref):
    degc = deg_ref[0, :, 0:1] + deg_ref[1, :, 0:1]
    e_new, b_new, s_new = _update_from_sums(
        _sum2(se_ref), _sum2(sb_ref), _sum2(ss_ref), degc,
        cb_ref, we_ref, be_ref, wb_ref, bb_ref)
    e_ref[...] = e_new
    b_ref[...] = b_new
    s_ref[...] = s_new


_GRID = _N // _BN

_SCALAR_SPEC = pl.BlockSpec((1, 1), lambda i: (0, 0), memory_space=pltpu.SMEM)
_ROW_SPEC = pl.BlockSpec((_BN, _D), lambda i: (i, 0))
_S_SPEC = pl.BlockSpec((_NC, _BN, _D), lambda i: (0, i, 0))
_DEG_SPEC = pl.BlockSpec((_NC, _BN, 16), lambda i: (0, i, 0))
_W_SPEC = pl.BlockSpec((_D, _D), lambda i: (0, 0))
_B_SPEC = pl.BlockSpec((1, _D), lambda i: (0, 0))

_ROW_SHAPE = jax.ShapeDtypeStruct((_N, _D), jnp.float32)


def _tc_prologue(cb, b, s, ws0, bs0):
    return pl.pallas_call(
        _tc_prologue_body,
        grid=(_GRID,),
        in_specs=[_SCALAR_SPEC, _ROW_SPEC, _ROW_SPEC, _W_SPEC, _B_SPEC],
        out_specs=[_ROW_SPEC, _ROW_SPEC],
        out_shape=[_ROW_SHAPE, _ROW_SHAPE],
    )(cb, b, s, ws0, bs0)


def _tc_mid(cb, se, sb, ss, deg2, we, be, wb, bb, wsn, bsn):
    return pl.pallas_call(
        _tc_mid_body,
        grid=(_GRID,),
        in_specs=[_SCALAR_SPEC, _S_SPEC, _S_SPEC, _S_SPEC, _DEG_SPEC,
                  _W_SPEC, _B_SPEC, _W_SPEC, _B_SPEC, _W_SPEC, _B_SPEC],
        out_specs=[_ROW_SPEC, _ROW_SPEC, _ROW_SPEC],
        out_shape=[_ROW_SHAPE, _ROW_SHAPE, _ROW_SHAPE],
    )(cb, se, sb, ss, deg2, we, be, wb, bb, wsn, bsn)


def _tc_epilogue(cb, se, sb, ss, deg2, we, be, wb, bb):
    return pl.pallas_call(
        _tc_epilogue_body,
        grid=(_GRID,),
        in_specs=[_SCALAR_SPEC, _S_SPEC, _S_SPEC, _S_SPEC, _DEG_SPEC,
                  _W_SPEC, _B_SPEC, _W_SPEC, _B_SPEC],
        out_specs=[_ROW_SPEC, _ROW_SPEC, _ROW_SPEC],
        out_shape=[_ROW_SHAPE, _ROW_SHAPE, _ROW_SHAPE],
    )(cb, se, sb, ss, deg2, we, be, wb, bb)


def _zero_slab(s, zsrc_hbm, acc_sh, r0):
    @pl.when(s < _NS - 1)
    def _():
        pltpu.sync_copy(zsrc_hbm.at[pl.ds(0, _SLAB)],
                        acc_sh.at[pl.ds(r0, _SLAB)])

    @pl.when(s == _NS - 1)
    def _():
        pltpu.sync_copy(zsrc_hbm, acc_sh.at[pl.ds(_LAST0, _SLAB_LAST)])


def _write_slab(s, acc_sh, out_hbm, c, r0):
    @pl.when(s < _NS - 1)
    def _():
        pltpu.sync_copy(acc_sh.at[pl.ds(r0, _SLAB)],
                        out_hbm.at[c, pl.ds(r0, _SLAB)])

    @pl.when(s == _NS - 1)
    def _():
        pltpu.sync_copy(acc_sh.at[pl.ds(_LAST0, _SLAB_LAST)],
                        out_hbm.at[c, pl.ds(_LAST0, _SLAB_LAST)])


_MESH = dict(core_axis_name="c", subcore_axis_name="s",
             num_cores=_NC, num_subcores=_NS)
_SC_PARAMS = pltpu.CompilerParams(use_tc_tiling_on_sc=False)


def _make_sc_segsum():
    mesh = plsc.VectorSubcoreMesh(**_MESH)

    scratch = [
        pltpu.VMEM_SHARED((_N, _D), jnp.float32),
        pltpu.VMEM((_NCH, _CH), jnp.int32),
        pltpu.VMEM((_NCH, _CH), jnp.int32),
        pltpu.VMEM((_CH, _D), jnp.float32),
        pltpu.VMEM((_CH, _D), jnp.float32),
        pltpu.SemaphoreType.DMA,
        pltpu.SemaphoreType.DMA,
    ]

    def body(t_hbm, src2_hbm, dst2_hbm, zrows_hbm, out_hbm,
             acc_sh, srcall, dstall, rows0, rows1, sem0, sem1):
        c = lax.axis_index("c")
        s = lax.axis_index("s")
        r0 = pl.multiple_of(s * _SLAB, 8)

        row0 = (c * _NS + s) * _NCH
        pltpu.sync_copy(src2_hbm.at[pl.ds(row0, _NCH)], srcall)
        pltpu.sync_copy(dst2_hbm.at[pl.ds(row0, _NCH)], dstall)
        _zero_slab(s, zrows_hbm, acc_sh, r0)
        plsc.subcore_barrier()

        def start(j, rows, sem):
            pltpu.async_copy(t_hbm.at[srcall.at[j]], rows, sem)

        def finish(j, rows, sem):
            pltpu.make_async_copy(t_hbm.at[srcall.at[j]], rows, sem).wait()
            pltpu.sync_copy(rows, acc_sh.at[dstall.at[j]], add=True)

        start(0, rows0, sem0)

        def pair(i, carry):
            j = 2 * i
            start(j + 1, rows1, sem1)
            finish(j, rows0, sem0)
            start(j + 2, rows0, sem0)
            finish(j + 1, rows1, sem1)
            return carry

        lax.fori_loop(0, (_NCH - 1) // 2, pair, 0)
        finish(_NCH - 1, rows0, sem0)

        plsc.subcore_barrier()
        _write_slab(s, acc_sh, out_hbm, c, r0)

    return pl.kernel(body,
                     out_type=jax.ShapeDtypeStruct((_NC, _N, _D), jnp.float32),
                     mesh=mesh, scratch_types=scratch,
                     compiler_params=_SC_PARAMS)


def _make_sc_deg():
    mesh = plsc.VectorSubcoreMesh(**_MESH)

    scratch = [
        pltpu.VMEM_SHARED((_N, 16), jnp.float32),
        pltpu.VMEM((_NCH, _CH), jnp.int32),
        pltpu.VMEM((_CH, 16), jnp.float32),
    ]

    def body(dst2_hbm, zdeg_hbm, ones_hbm, deg_hbm,
             deg_sh, dstall, onesb):
        c = lax.axis_index("c")
        s = lax.axis_index("s")
        r0 = pl.multiple_of(s * _SLAB, 8)

        row0 = (c * _NS + s) * _NCH
        pltpu.sync_copy(dst2_hbm.at[pl.ds(row0, _NCH)], dstall)
        pltpu.sync_copy(ones_hbm, onesb)
        _zero_slab(s, zdeg_hbm, deg_sh, r0)
        plsc.subcore_barrier()

        def chunk(j, carry):
            pltpu.sync_copy(onesb, deg_sh.at[dstall.at[j]], add=True)
            return carry

        lax.fori_loop(0, _NCH, chunk, 0)

        plsc.subcore_barrier()
        _write_slab(s, deg_sh, deg_hbm, c, r0)

    return pl.kernel(body,
                     out_type=jax.ShapeDtypeStruct((_NC, _N, 16), jnp.float32),
                     mesh=mesh, scratch_types=scratch,
                     compiler_params=_SC_PARAMS)


def kernel(e_emb, b_emb, s_emb, edge_index, We, be, Wb, bb, Ws, bs,
           b_curvature, s_curvature):
    sc_seg = _make_sc_segsum()
    sc_deg = _make_sc_deg()
    src2 = edge_index[0].astype(jnp.int32).reshape(_E // _CH, _CH)
    dst2 = edge_index[1].astype(jnp.int32).reshape(_E // _CH, _CH)
    cb = b_curvature.reshape(1, 1)
    be2 = be.reshape(2, 1, _D)
    bb2 = bb.reshape(2, 1, _D)
    bs2 = bs.reshape(2, 1, _D)
    zrows = jnp.zeros((_SLAB_LAST, _D), jnp.float32)
    zdeg = jnp.zeros((_SLAB_LAST, 16), jnp.float32)
    ones = jnp.ones((_CH, 16), jnp.float32)

    tb0, ts0 = _tc_prologue(cb, b_emb, s_emb, Ws[0], bs2[0])
    deg2 = sc_deg(dst2, zdeg, ones)
    se0 = sc_seg(e_emb, src2, dst2, zrows)
    sb0 = sc_seg(tb0, src2, dst2, zrows)
    ss0 = sc_seg(ts0, src2, dst2, zrows)
    e1, tb1, ts1 = _tc_mid(cb, se0, sb0, ss0, deg2,
                           We[0], be2[0], Wb[0], bb2[0], Ws[1], bs2[1])
    se1 = sc_seg(e1, src2, dst2, zrows)
    sb1 = sc_seg(tb1, src2, dst2, zrows)
    ss1 = sc_seg(ts1, src2, dst2, zrows)
    e2, b2, s2 = _tc_epilogue(cb, se1, sb1, ss1, deg2,
                              We[1], be2[1], Wb[1], bb2[1])
    return (e2, b2, s2)


# speedup vs baseline: 7.9308x; 1.1471x over previous
import jax
import jax.numpy as jnp
from jax import lax
from jax.experimental import pallas as pl
from jax.experimental.pallas import tpu as pltpu
from jax.experimental.pallas import tpu_sc as plsc

_N = 10000
_D = 128
_NC = 2
_NS = 16
_E = 320000
_PT = _E // (_NC * _NS)
_CH = 80
_NCH = _PT // _CH
_SLAB = 624
_SLAB_LAST = _N - (_NS - 1) * _SLAB
_LAST0 = (_NS - 1) * _SLAB
_BN = 1000
_EPS = 1e-12


def _rownorm(x):
    return jnp.sqrt(jnp.sum(x * x, axis=1, keepdims=True))


def _matmul_t(x, w):
    return lax.dot_general(x, w, (((1,), (1,)), ((), ())),
                           preferred_element_type=jnp.float32)


def _log_map_table(b, sc):
    bn = _rownorm(b)
    at = 0.5 * jnp.log((1.0 + sc * bn) / (1.0 - sc * bn))
    return (2.0 / sc) * at * b / bn


def _sphere_table(s, ws, bs):
    ns = s / jnp.maximum(_rownorm(s), _EPS)
    ts0 = _matmul_t(ns, ws) + bs
    return ts0 / jnp.maximum(_rownorm(ts0), _EPS)


def _tc_prologue_body(cb_ref, b_ref, s_ref, ws_ref, bs_ref, tb_ref, ts_ref):
    sc = jnp.sqrt(cb_ref[0, 0])
    tb_ref[...] = _log_map_table(b_ref[...], sc)
    ts_ref[...] = _sphere_table(s_ref[...], ws_ref[...], bs_ref[...])


def _update_from_sums(se, sb, ss, degc, cb_ref, we_ref, be_ref, wb_ref, bb_ref):
    sc = jnp.sqrt(cb_ref[0, 0])
    invd = 1.0 / jnp.maximum(degc, 1.0)
    mask = jnp.minimum(degc, 1.0)
    ue = _matmul_t(se * invd, we_ref[...]) + be_ref[...] * mask
    e_new = jnp.where(ue >= 0, ue, 0.2 * ue)
    nb = _matmul_t(sb * invd, wb_ref[...]) + bb_ref[...] * mask
    nbn = _rownorm(nb)
    b_new = jnp.tanh(sc * nbn * 0.5) * nb / (sc * nbn)
    mts = ss * invd
    s_new = mts / jnp.maximum(_rownorm(mts), _EPS)
    return e_new, b_new, s_new


def _sum2(ref):
    return ref[0] + ref[1]


def _tc_mid_body(cb_ref, se_ref, sb_ref, ss_ref, deg_ref, we_ref, be_ref,
                 wb_ref, bb_ref, wsn_ref, bsn_ref, e_ref, tb_ref, ts_ref):
    degc = deg_ref[0, :, 0:1] + deg_ref[1, :, 0:1]
    e_new, b_new, s_new = _update_from_sums(
        _sum2(se_ref), _sum2(sb_ref), _sum2(ss_ref), degc,
        cb_ref, we_ref, be_ref, wb_ref, bb_ref)
    sc = jnp.sqrt(cb_ref[0, 0])
    e_ref[...] = e_new
    tb_ref[...] = _log_map_table(b_new, sc)
    ts_ref[...] = _sphere_table(s_new, wsn_ref[...], bsn_ref[...])


def _tc_epilogue_body(cb_ref, se_ref, sb_ref, ss_ref, deg_ref, we_ref, be_ref,
                      wb_ref, bb_ref, e_ref, b_ref, s_ref):
    degc = deg_ref[0, :, 0:1] + deg_ref[1, :, 0:1]
    e_new, b_new, s_new = _update_from_sums(
        _sum2(se_ref), _sum2(sb_ref), _sum2(ss_ref), degc,
        cb_ref, we_ref, be_ref, wb_ref, bb_ref)
    e_ref[...] = e_new
    b_ref[...] = b_new
    s_ref[...] = s_new


_GRID = _N // _BN

_SCALAR_SPEC = pl.BlockSpec((1, 1), lambda i: (0, 0), memory_space=pltpu.SMEM)
_ROW_SPEC = pl.BlockSpec((_BN, _D), lambda i: (i, 0))
_S_SPEC = pl.BlockSpec((_NC, _BN, _D), lambda i: (0, i, 0))
_DEG_SPEC = pl.BlockSpec((_NC, _BN, 16), lambda i: (0, i, 0))
_W_SPEC = pl.BlockSpec((_D, _D), lambda i: (0, 0))
_B_SPEC = pl.BlockSpec((1, _D), lambda i: (0, 0))

_ROW_SHAPE = jax.ShapeDtypeStruct((_N, _D), jnp.float32)


def _tc_prologue(cb, b, s, ws0, bs0):
    return pl.pallas_call(
        _tc_prologue_body,
        grid=(_GRID,),
        in_specs=[_SCALAR_SPEC, _ROW_SPEC, _ROW_SPEC, _W_SPEC, _B_SPEC],
        out_specs=[_ROW_SPEC, _ROW_SPEC],
        out_shape=[_ROW_SHAPE, _ROW_SHAPE],
    )(cb, b, s, ws0, bs0)


def _tc_mid(cb, se, sb, ss, deg2, we, be, wb, bb, wsn, bsn):
    return pl.pallas_call(
        _tc_mid_body,
        grid=(_GRID,),
        in_specs=[_SCALAR_SPEC, _S_SPEC, _S_SPEC, _S_SPEC, _DEG_SPEC,
                  _W_SPEC, _B_SPEC, _W_SPEC, _B_SPEC, _W_SPEC, _B_SPEC],
        out_specs=[_ROW_SPEC, _ROW_SPEC, _ROW_SPEC],
        out_shape=[_ROW_SHAPE, _ROW_SHAPE, _ROW_SHAPE],
    )(cb, se, sb, ss, deg2, we, be, wb, bb, wsn, bsn)


def _tc_epilogue(cb, se, sb, ss, deg2, we, be, wb, bb):
    return pl.pallas_call(
        _tc_epilogue_body,
        grid=(_GRID,),
        in_specs=[_SCALAR_SPEC, _S_SPEC, _S_SPEC, _S_SPEC, _DEG_SPEC,
                  _W_SPEC, _B_SPEC, _W_SPEC, _B_SPEC],
        out_specs=[_ROW_SPEC, _ROW_SPEC, _ROW_SPEC],
        out_shape=[_ROW_SHAPE, _ROW_SHAPE, _ROW_SHAPE],
    )(cb, se, sb, ss, deg2, we, be, wb, bb)


def _zero_slab(s, zsrc_hbm, acc_sh, r0):
    @pl.when(s < _NS - 1)
    def _():
        pltpu.sync_copy(zsrc_hbm.at[pl.ds(0, _SLAB)],
                        acc_sh.at[pl.ds(r0, _SLAB)])

    @pl.when(s == _NS - 1)
    def _():
        pltpu.sync_copy(zsrc_hbm, acc_sh.at[pl.ds(_LAST0, _SLAB_LAST)])


def _write_slab(s, acc_sh, out_hbm, c, r0):
    @pl.when(s < _NS - 1)
    def _():
        pltpu.sync_copy(acc_sh.at[pl.ds(r0, _SLAB)],
                        out_hbm.at[c, pl.ds(r0, _SLAB)])

    @pl.when(s == _NS - 1)
    def _():
        pltpu.sync_copy(acc_sh.at[pl.ds(_LAST0, _SLAB_LAST)],
                        out_hbm.at[c, pl.ds(_LAST0, _SLAB_LAST)])


_MESH = dict(core_axis_name="c", subcore_axis_name="s",
             num_cores=_NC, num_subcores=_NS)
_SC_PARAMS = pltpu.CompilerParams(use_tc_tiling_on_sc=False)


def _make_sc_segsum():
    mesh = plsc.VectorSubcoreMesh(**_MESH)

    scratch = [
        pltpu.VMEM_SHARED((_N, _D), jnp.float32),
        pltpu.VMEM((_NCH, _CH), jnp.int32),
        pltpu.VMEM((_NCH, _CH), jnp.int32),
        pltpu.VMEM((_CH, _D), jnp.float32),
        pltpu.VMEM((_CH, _D), jnp.float32),
        pltpu.VMEM((_CH, _D), jnp.float32),
        pltpu.SemaphoreType.DMA,
        pltpu.SemaphoreType.DMA,
        pltpu.SemaphoreType.DMA,
        pltpu.SemaphoreType.DMA,
        pltpu.SemaphoreType.DMA,
        pltpu.SemaphoreType.DMA,
    ]

    def body(t_hbm, src2_hbm, dst2_hbm, zrows_hbm, out_hbm,
             acc_sh, srcall, dstall, rows0, rows1, rows2,
             g0, g1, g2, s0, s1, s2):
        c = lax.axis_index("c")
        s = lax.axis_index("s")
        r0 = pl.multiple_of(s * _SLAB, 8)
        bufs = ((rows0, g0, s0), (rows1, g1, s1), (rows2, g2, s2))

        row0 = (c * _NS + s) * _NCH
        pltpu.sync_copy(src2_hbm.at[pl.ds(row0, _NCH)], srcall)
        pltpu.sync_copy(dst2_hbm.at[pl.ds(row0, _NCH)], dstall)
        _zero_slab(s, zrows_hbm, acc_sh, r0)
        plsc.subcore_barrier()

        def sg(j, k):
            rows, gsem, _ = bufs[k]
            pltpu.async_copy(t_hbm.at[srcall.at[j]], rows, gsem)

        def wg(j, k):
            rows, gsem, _ = bufs[k]
            pltpu.make_async_copy(t_hbm.at[srcall.at[j]], rows, gsem).wait()

        def ss(j, k):
            rows, _, ssem = bufs[k]
            pltpu.async_copy(rows, acc_sh.at[dstall.at[j]], ssem, add=True)

        def ws(k):
            rows, _, ssem = bufs[k]
            pltpu.make_async_copy(rows, acc_sh.at[dstall.at[0]], ssem).wait()

        sg(0, 0)
        sg(1, 1)

        def step(i, carry):
            for k in range(3):
                @pl.when(i % 3 == k)
                def _(k=k):
                    wg(i, k)
                    ss(i, k)
            for k in range(3):
                @pl.when(jnp.logical_and(i >= 1, (i - 1) % 3 == k))
                def _(k=k):
                    ws(k)
            for k in range(3):
                @pl.when(jnp.logical_and(i + 2 < _NCH, (i + 2) % 3 == k))
                def _(k=k):
                    sg(i + 2, k)
            return carry

        lax.fori_loop(0, _NCH, step, 0)
        ws((_NCH - 1) % 3)

        plsc.subcore_barrier()
        _write_slab(s, acc_sh, out_hbm, c, r0)

    return pl.kernel(body,
                     out_type=jax.ShapeDtypeStruct((_NC, _N, _D), jnp.float32),
                     mesh=mesh, scratch_types=scratch,
                     compiler_params=_SC_PARAMS)


def _make_sc_deg():
    mesh = plsc.VectorSubcoreMesh(**_MESH)

    scratch = [
        pltpu.VMEM_SHARED((_N, 16), jnp.float32),
        pltpu.VMEM((_NCH, _CH), jnp.int32),
        pltpu.VMEM((_CH, 16), jnp.float32),
    ]

    def body(dst2_hbm, zdeg_hbm, ones_hbm, deg_hbm,
             deg_sh, dstall, onesb):
        c = lax.axis_index("c")
        s = lax.axis_index("s")
        r0 = pl.multiple_of(s * _SLAB, 8)

        row0 = (c * _NS + s) * _NCH
        pltpu.sync_copy(dst2_hbm.at[pl.ds(row0, _NCH)], dstall)
        pltpu.sync_copy(ones_hbm, onesb)
        _zero_slab(s, zdeg_hbm, deg_sh, r0)
        plsc.subcore_barrier()

        def chunk(j, carry):
            pltpu.sync_copy(onesb, deg_sh.at[dstall.at[j]], add=True)
            return carry

        lax.fori_loop(0, _NCH, chunk, 0)

        plsc.subcore_barrier()
        _write_slab(s, deg_sh, deg_hbm, c, r0)

    return pl.kernel(body,
                     out_type=jax.ShapeDtypeStruct((_NC, _N, 16), jnp.float32),
                     mesh=mesh, scratch_types=scratch,
                     compiler_params=_SC_PARAMS)


def kernel(e_emb, b_emb, s_emb, edge_index, We, be, Wb, bb, Ws, bs,
           b_curvature, s_curvature):
    sc_seg = _make_sc_segsum()
    sc_deg = _make_sc_deg()
    src2 = edge_index[0].astype(jnp.int32).reshape(_E // _CH, _CH)
    dst2 = edge_index[1].astype(jnp.int32).reshape(_E // _CH, _CH)
    cb = b_curvature.reshape(1, 1)
    be2 = be.reshape(2, 1, _D)
    bb2 = bb.reshape(2, 1, _D)
    bs2 = bs.reshape(2, 1, _D)
    zrows = jnp.zeros((_SLAB_LAST, _D), jnp.float32)
    zdeg = jnp.zeros((_SLAB_LAST, 16), jnp.float32)
    ones = jnp.ones((_CH, 16), jnp.float32)

    tb0, ts0 = _tc_prologue(cb, b_emb, s_emb, Ws[0], bs2[0])
    deg2 = sc_deg(dst2, zdeg, ones)
    se0 = sc_seg(e_emb, src2, dst2, zrows)
    sb0 = sc_seg(tb0, src2, dst2, zrows)
    ss0 = sc_seg(ts0, src2, dst2, zrows)
    e1, tb1, ts1 = _tc_mid(cb, se0, sb0, ss0, deg2,
                           We[0], be2[0], Wb[0], bb2[0], Ws[1], bs2[1])
    se1 = sc_seg(e1, src2, dst2, zrows)
    sb1 = sc_seg(tb1, src2, dst2, zrows)
    ss1 = sc_seg(ts1, src2, dst2, zrows)
    e2, b2, s2 = _tc_epilogue(cb, se1, sb1, ss1, deg2,
                              We[1], be2[1], Wb[1], bb2[1])
    return (e2, b2, s2)
